# baseline (speedup 1.0000x reference)
import jax
import jax.numpy as jnp
from jax.experimental import pallas as pl
from jax.experimental.pallas import tpu as pltpu

B, H, S, D = 8, 16, 4096, 128
Q = 16
BH = B * H
G = 4


def _body(pos_ref, val_ref, c_ref, o_ref):
    o_ref[...] = c_ref[...]
    for q in range(Q):
        p = pos_ref[q]
        m = q
        for r in range(q + 1, Q):
            m = jnp.where(pos_ref[r] == p, r, m)
        for g in range(G):
            o_ref[g, pl.ds(p, 1), :] = val_ref[g, pl.ds(m, 1), :]


def _update(pos, val, cache):
    cache_spec = pl.BlockSpec((G, S, D), lambda i: (i, 0, 0))
    val_spec = pl.BlockSpec((G, Q, D), lambda i: (i, 0, 0))
    return pl.pallas_call(
        _body,
        grid=(BH // G,),
        in_specs=[
            pl.BlockSpec(memory_space=pltpu.SMEM),
            val_spec,
            cache_spec,
        ],
        out_specs=cache_spec,
        out_shape=jax.ShapeDtypeStruct((BH, S, D), jnp.float32),
        compiler_params=pltpu.CompilerParams(
            dimension_semantics=("arbitrary",),
        ),
    )(pos, val, cache)


def kernel(input_pos, k_val, v_val, k_cache, v_cache):
    ko = _update(input_pos, k_val.reshape(BH, Q, D), k_cache.reshape(BH, S, D))
    vo = _update(input_pos, v_val.reshape(BH, Q, D), v_cache.reshape(BH, S, D))
    return ko.reshape(B, H, S, D), vo.reshape(B, H, S, D)


# speedup vs baseline: 1.0009x; 1.0009x over previous
import jax
import jax.numpy as jnp
from jax.experimental import pallas as pl
from jax.experimental.pallas import tpu as pltpu

B, H, S, D = 8, 16, 4096, 128
Q = 16
BH = B * H
G = 2


def _body(pos_ref, kval_ref, vval_ref, kc_ref, vc_ref, ko_ref, vo_ref):
    ko_ref[...] = kc_ref[...]
    vo_ref[...] = vc_ref[...]
    for q in range(Q):
        p = pos_ref[q]
        m = q
        for r in range(q + 1, Q):
            m = jnp.where(pos_ref[r] == p, r, m)
        for g in range(G):
            ko_ref[g, pl.ds(p, 1), :] = kval_ref[g, pl.ds(m, 1), :]
            vo_ref[g, pl.ds(p, 1), :] = vval_ref[g, pl.ds(m, 1), :]


def kernel(input_pos, k_val, v_val, k_cache, v_cache):
    kc = k_cache.reshape(BH, S, D)
    vc = v_cache.reshape(BH, S, D)
    kv = k_val.reshape(BH, Q, D)
    vv = v_val.reshape(BH, Q, D)
    cache_spec = pl.BlockSpec((G, S, D), lambda i: (i, 0, 0))
    val_spec = pl.BlockSpec((G, Q, D), lambda i: (i, 0, 0))
    ko, vo = pl.pallas_call(
        _body,
        grid=(BH // G,),
        in_specs=[
            pl.BlockSpec(memory_space=pltpu.SMEM),
            val_spec,
            val_spec,
            cache_spec,
            cache_spec,
        ],
        out_specs=[cache_spec, cache_spec],
        out_shape=[
            jax.ShapeDtypeStruct((BH, S, D), jnp.float32),
            jax.ShapeDtypeStruct((BH, S, D), jnp.float32),
        ],
        compiler_params=pltpu.CompilerParams(
            dimension_semantics=("parallel",),
        ),
    )(input_pos, kv, vv, kc, vc)
    return ko.reshape(B, H, S, D), vo.reshape(B, H, S, D)
